# baseline (speedup 1.0000x reference)
import functools

import jax
import jax.numpy as jnp
from jax import lax
from jax.experimental import pallas as pl
from jax.experimental.pallas import tpu as pltpu
from jax.experimental.pallas import tpu_sc as plsc

_BN = 4000
_CH = 2560
_NW = 32


def _sc_body(anc_hbm, ann_hbm, clsf_hbm, out_hbm,
             ax1v, ay1v, ax2v, ay2v, annv,
             omaxv, og1v, og2v, og3v, og4v, idxv, gv, sem,
             *, m, nmax, c):
    wid = lax.axis_index("s") * 2 + lax.axis_index("c")
    b = wid // 8
    pltpu.sync_copy(anc_hbm.at[wid, 0], ax1v)
    pltpu.sync_copy(anc_hbm.at[wid, 1], ay1v)
    pltpu.sync_copy(anc_hbm.at[wid, 2], ax2v)
    pltpu.sync_copy(anc_hbm.at[wid, 3], ay2v)
    pltpu.sync_copy(ann_hbm.at[b], annv)

    def field(k):
        vs = [annv[pl.ds(k * m + h * 16, 16)] for h in range(m // 16)]
        return [vs[mm // 16][mm % 16] for mm in range(m)]

    bx1s = field(0)
    by1s = field(1)
    bx2s = field(2)
    by2s = field(3)
    areab = [(bx2s[mm] - bx1s[mm]) * (by2s[mm] - by1s[mm]) for mm in range(m)]
    per_w = nmax // _NW
    iota16 = lax.iota(jnp.int32, 16)

    def group(j, carry):
        sl = pl.ds(j * 16, 16)
        ax1 = ax1v[sl]
        ay1 = ay1v[sl]
        ax2 = ax2v[sl]
        ay2 = ay2v[sl]
        area_a = (ax2 - ax1) * (ay2 - ay1)
        best = jnp.full((16,), -2.0, jnp.float32)
        barg = jnp.zeros((16,), jnp.int32)
        for mm in range(m):
            iw = jnp.maximum(jnp.minimum(ax2, bx2s[mm]) - jnp.maximum(ax1, bx1s[mm]), 0.0)
            ih = jnp.maximum(jnp.minimum(ay2, by2s[mm]) - jnp.maximum(ay1, by1s[mm]), 0.0)
            inter = iw * ih
            union = jnp.maximum(area_a + areab[mm] - inter, 1e-8)
            iou = inter / union
            upd = iou > best
            barg = jnp.where(upd, mm, barg)
            best = jnp.where(upd, iou, best)
        omaxv[sl] = best
        og1v[sl] = plsc.load_gather(annv, [barg])
        og2v[sl] = plsc.load_gather(annv, [barg + m])
        og3v[sl] = plsc.load_gather(annv, [barg + 2 * m])
        og4v[sl] = plsc.load_gather(annv, [barg + 3 * m])
        lab = plsc.load_gather(annv, [barg + 4 * m]).astype(jnp.int32)
        nglob = jnp.minimum(wid * per_w + j * 16 + iota16, nmax - 1)
        idxv[sl] = nglob * c + lab
        return carry

    lax.fori_loop(0, _CH // 16, group, 0)

    copies = [
        pltpu.async_copy(clsf_hbm.at[idxv.at[pl.ds(t * 128, 128)]],
                         gv.at[pl.ds(t * 128, 128)], sem)
        for t in range(_CH // 128)
    ]
    for cp in copies:
        cp.wait()

    pltpu.sync_copy(omaxv, out_hbm.at[wid, 0])
    pltpu.sync_copy(og1v, out_hbm.at[wid, 1])
    pltpu.sync_copy(og2v, out_hbm.at[wid, 2])
    pltpu.sync_copy(og3v, out_hbm.at[wid, 3])
    pltpu.sync_copy(og4v, out_hbm.at[wid, 4])
    pltpu.sync_copy(gv, out_hbm.at[wid, 5])


def _sc_match(anc_pad, ann_flat, cls_flat, m, nmax, c):
    mesh = plsc.VectorSubcoreMesh(core_axis_name="c", subcore_axis_name="s")
    f32 = jnp.float32
    kern = functools.partial(
        pl.kernel,
        mesh=mesh,
        compiler_params=pltpu.CompilerParams(needs_layout_passes=False),
        out_type=jax.ShapeDtypeStruct((_NW, 6, _CH), f32),
        scratch_types=[
            pltpu.VMEM((_CH,), f32), pltpu.VMEM((_CH,), f32),
            pltpu.VMEM((_CH,), f32), pltpu.VMEM((_CH,), f32),
            pltpu.VMEM((5 * m,), f32),
            pltpu.VMEM((_CH,), f32), pltpu.VMEM((_CH,), f32),
            pltpu.VMEM((_CH,), f32), pltpu.VMEM((_CH,), f32),
            pltpu.VMEM((_CH,), f32),
            pltpu.VMEM((_CH,), jnp.int32), pltpu.VMEM((_CH,), f32),
            pltpu.SemaphoreType.DMA,
        ],
    )(functools.partial(_sc_body, m=m, nmax=nmax, c=c))
    return kern(anc_pad, ann_flat, cls_flat)


def _tca_body(cls_ref, row_o, *, bn, c):
    cls = jnp.clip(cls_ref[0], 0.0001, 1.0 - 0.0001)
    logm = jnp.log(1.0 - cls)
    nege = (0.75 * (cls * cls)) * logm
    row_o[0] = jnp.sum(nege, axis=1, keepdims=True)


def _tcb_body(row_ref, reg_ref, anc_ref, sc_ref, cls_o, npos_o, reg_o, *, bn):
    i = pl.program_id(1)

    @pl.when(i == 0)
    def _init():
        cls_o[...] = jnp.zeros_like(cls_o)
        npos_o[...] = jnp.zeros_like(npos_o)
        reg_o[...] = jnp.zeros_like(reg_o)

    sc = sc_ref[0, 0]
    iou_max = sc[0:1, :]
    gx1 = sc[1:2, :]
    gy1 = sc[2:3, :]
    gx2 = sc[3:4, :]
    gy2 = sc[4:5, :]
    g = sc[5:6, :]
    anc = anc_ref[0, 0]
    ax1 = anc[0:1, :]
    ay1 = anc[1:2, :]
    ax2 = anc[2:3, :]
    ay2 = anc[3:4, :]
    row_neg = row_ref[0, 0]

    pos = iou_max >= 0.5
    keep = jnp.logical_or(iou_max < 0.4, pos)
    npos_part = jnp.sum(pos.astype(jnp.float32))

    aw = ax2 - ax1
    ah = ay2 - ay1
    acx = ax1 + 0.5 * aw
    acy = ay1 + 0.5 * ah
    gw = gx2 - gx1
    gh = gy2 - gy1
    gcx = gx1 + 0.5 * gw
    gcy = gy1 + 0.5 * gh
    gw = jnp.maximum(gw, 1.0)
    gh = jnp.maximum(gh, 1.0)
    t0 = ((gcx - acx) / aw) / 0.1
    t1 = ((gcy - acy) / ah) / 0.1
    t2 = jnp.log(gw / aw) / 0.2
    t3 = jnp.log(gh / ah) / 0.2
    reg = reg_ref[0, 0]
    d0 = jnp.abs(t0 - reg[0:1, :])
    d1 = jnp.abs(t1 - reg[1:2, :])
    d2 = jnp.abs(t2 - reg[2:3, :])
    d3 = jnp.abs(t3 - reg[3:4, :])

    def smooth_l1(d):
        return jnp.where(d < 1.0 / 9.0, 0.5 * 9.0 * (d * d), d - 0.5 / 9.0)

    rl = smooth_l1(d0) + smooth_l1(d1) + smooth_l1(d2) + smooth_l1(d3)
    reg_part = jnp.sum(jnp.where(pos, rl, 0.0))

    gc = jnp.clip(g, 0.0001, 1.0 - 0.0001)
    pos_e = (0.25 * ((1.0 - gc) * (1.0 - gc))) * (-jnp.log(gc))
    neg_e = (0.75 * (gc * gc)) * (-jnp.log(1.0 - gc))
    corr = jnp.where(pos, pos_e - neg_e, 0.0)
    tot_neg = jnp.sum(jnp.where(keep, row_neg, 0.0))
    cls_part = jnp.sum(corr) - tot_neg

    cls_o[...] += jnp.full(cls_o.shape, cls_part, jnp.float32)
    npos_o[...] += jnp.full(npos_o.shape, npos_part, jnp.float32)
    reg_o[...] += jnp.full(reg_o.shape, reg_part, jnp.float32)


@jax.jit
def kernel(classifications, regressions, anchors, annotations):
    b, n, c = classifications.shape
    m = annotations.shape[1]
    bn = _BN
    nb = n // bn
    per_w = (b * n) // _NW

    anc_pad = jnp.pad(anchors.reshape(_NW, per_w, 4),
                      ((0, 0), (0, _CH - per_w), (0, 0))).transpose(0, 2, 1)
    ann_sc = annotations.transpose(0, 2, 1).reshape(b, 5 * m)
    cls_flat = classifications.reshape(b * n * c)
    scm = _sc_match(anc_pad, ann_sc, cls_flat, m, b * n, c)
    scm = scm[:, :, :per_w].reshape(b, _NW // b, 6, per_w)
    scm = scm.transpose(0, 2, 1, 3).reshape(b, 6, nb, bn)
    scm = scm.transpose(0, 2, 1, 3)

    rows = pl.pallas_call(
        functools.partial(_tca_body, bn=bn, c=c),
        grid=(b, nb),
        in_specs=[pl.BlockSpec((1, bn, c), lambda bb, ii: (bb, ii, 0))],
        out_specs=pl.BlockSpec((1, bn, 1), lambda bb, ii: (bb, ii, 0)),
        out_shape=jax.ShapeDtypeStruct((b, n, 1), jnp.float32),
    )(classifications)
    rows4 = rows.reshape(b, nb, 1, bn)

    anc_t = anchors.reshape(b, nb, bn, 4).transpose(0, 1, 3, 2)
    reg_t = regressions.reshape(b, nb, bn, 4).transpose(0, 1, 3, 2)

    out_sds = jax.ShapeDtypeStruct((b, 1, 128), jnp.float32)
    cls_s, npos, reg_s = pl.pallas_call(
        functools.partial(_tcb_body, bn=bn),
        grid=(b, nb),
        in_specs=[
            pl.BlockSpec((1, 1, 1, bn), lambda bb, ii: (bb, ii, 0, 0)),
            pl.BlockSpec((1, 1, 4, bn), lambda bb, ii: (bb, ii, 0, 0)),
            pl.BlockSpec((1, 1, 4, bn), lambda bb, ii: (bb, ii, 0, 0)),
            pl.BlockSpec((1, 1, 6, bn), lambda bb, ii: (bb, ii, 0, 0)),
        ],
        out_specs=[
            pl.BlockSpec((1, 1, 128), lambda bb, ii: (bb, 0, 0)),
            pl.BlockSpec((1, 1, 128), lambda bb, ii: (bb, 0, 0)),
            pl.BlockSpec((1, 1, 128), lambda bb, ii: (bb, 0, 0)),
        ],
        out_shape=[out_sds, out_sds, out_sds],
    )(rows4, reg_t, anc_t, scm)

    cls_s = cls_s[:, 0, 0]
    npos = npos[:, 0, 0]
    reg_s = reg_s[:, 0, 0]
    cls_tot = jnp.where(npos > 0, cls_s / jnp.maximum(npos, 1.0), 0.0)
    reg_tot = jnp.where(npos > 0, reg_s / jnp.maximum(4.0 * npos, 1.0), 0.0)
    return jnp.mean(cls_tot), jnp.mean(reg_tot)


# speedup vs baseline: 1.4884x; 1.4884x over previous
import functools

import jax
import jax.numpy as jnp
from jax import lax
from jax.experimental import pallas as pl
from jax.experimental.pallas import tpu as pltpu
from jax.experimental.pallas import tpu_sc as plsc

_BN = 4000
_CH = 2512
_NW = 32


def _sc_body(anc_hbm, ann_hbm, out_hbm,
             ax1v, ay1v, ax2v, ay2v, annv,
             omaxv, og1v, og2v, og3v, og4v, olabv, *, m):
    wid = lax.axis_index("s") * 2 + lax.axis_index("c")
    b = wid // 8
    pltpu.sync_copy(anc_hbm.at[wid, 0], ax1v)
    pltpu.sync_copy(anc_hbm.at[wid, 1], ay1v)
    pltpu.sync_copy(anc_hbm.at[wid, 2], ax2v)
    pltpu.sync_copy(anc_hbm.at[wid, 3], ay2v)
    pltpu.sync_copy(ann_hbm.at[b], annv)

    def field(k):
        vs = [annv[pl.ds(k * m + h * 16, 16)] for h in range(m // 16)]
        return [vs[mm // 16][mm % 16] for mm in range(m)]

    bx1s = field(0)
    by1s = field(1)
    bx2s = field(2)
    by2s = field(3)
    labs = field(4)
    areab = [(bx2s[mm] - bx1s[mm]) * (by2s[mm] - by1s[mm]) for mm in range(m)]

    @plsc.parallel_loop(0, _CH // 16, unroll=2)
    def group(j):
        sl = pl.ds(j * 16, 16)
        ax1 = ax1v[sl]
        ay1 = ay1v[sl]
        ax2 = ax2v[sl]
        ay2 = ay2v[sl]
        area_a = (ax2 - ax1) * (ay2 - ay1)
        best = jnp.full((16,), -2.0, jnp.float32)
        barg = jnp.zeros((16,), jnp.int32)
        for mm in range(m):
            iw = jnp.maximum(jnp.minimum(ax2, bx2s[mm]) - jnp.maximum(ax1, bx1s[mm]), 0.0)
            ih = jnp.maximum(jnp.minimum(ay2, by2s[mm]) - jnp.maximum(ay1, by1s[mm]), 0.0)
            inter = iw * ih
            union = jnp.maximum(area_a + areab[mm] - inter, 1e-8)
            iou = inter / union
            iou = jnp.where(labs[mm] != -1.0, iou, -1.0)
            upd = iou > best
            barg = jnp.where(upd, mm, barg)
            best = jnp.where(upd, iou, best)
        omaxv[sl] = best
        og1v[sl] = plsc.load_gather(annv, [barg])
        og2v[sl] = plsc.load_gather(annv, [barg + m])
        og3v[sl] = plsc.load_gather(annv, [barg + 2 * m])
        og4v[sl] = plsc.load_gather(annv, [barg + 3 * m])
        olabv[sl] = plsc.load_gather(annv, [barg + 4 * m])

    pltpu.sync_copy(omaxv, out_hbm.at[wid, 0])
    pltpu.sync_copy(og1v, out_hbm.at[wid, 1])
    pltpu.sync_copy(og2v, out_hbm.at[wid, 2])
    pltpu.sync_copy(og3v, out_hbm.at[wid, 3])
    pltpu.sync_copy(og4v, out_hbm.at[wid, 4])
    pltpu.sync_copy(olabv, out_hbm.at[wid, 5])


def _sc_match(anc_pad, annotations, m):
    mesh = plsc.VectorSubcoreMesh(core_axis_name="c", subcore_axis_name="s")
    f32 = jnp.float32
    kern = functools.partial(
        pl.kernel,
        mesh=mesh,
        compiler_params=pltpu.CompilerParams(needs_layout_passes=False),
        out_type=jax.ShapeDtypeStruct((_NW, 6, _CH), f32),
        scratch_types=[
            pltpu.VMEM((_CH,), f32), pltpu.VMEM((_CH,), f32),
            pltpu.VMEM((_CH,), f32), pltpu.VMEM((_CH,), f32),
            pltpu.VMEM((5 * m,), f32),
            pltpu.VMEM((_CH,), f32), pltpu.VMEM((_CH,), f32),
            pltpu.VMEM((_CH,), f32), pltpu.VMEM((_CH,), f32),
            pltpu.VMEM((_CH,), f32), pltpu.VMEM((_CH,), f32),
        ],
    )(functools.partial(_sc_body, m=m))
    return kern(anc_pad, annotations)


def _tc_body(cls_ref, reg_ref, anc_ref, sc_ref, cls_o, npos_o, reg_o, *, bn, c):
    i = pl.program_id(1)

    @pl.when(i == 0)
    def _init():
        cls_o[...] = jnp.zeros_like(cls_o)
        npos_o[...] = jnp.zeros_like(npos_o)
        reg_o[...] = jnp.zeros_like(reg_o)

    sc = sc_ref[0, 0]
    iou_max = sc[0:1, :]
    gx1 = sc[1:2, :]
    gy1 = sc[2:3, :]
    gx2 = sc[3:4, :]
    gy2 = sc[4:5, :]
    glab = sc[5:6, :]
    anc = anc_ref[0, 0]
    ax1 = anc[0:1, :]
    ay1 = anc[1:2, :]
    ax2 = anc[2:3, :]
    ay2 = anc[3:4, :]

    pos = iou_max >= 0.5
    keep = jnp.logical_or(iou_max < 0.4, pos)
    npos_part = jnp.sum(pos.astype(jnp.float32))

    aw = ax2 - ax1
    ah = ay2 - ay1
    acx = ax1 + 0.5 * aw
    acy = ay1 + 0.5 * ah
    gw = gx2 - gx1
    gh = gy2 - gy1
    gcx = gx1 + 0.5 * gw
    gcy = gy1 + 0.5 * gh
    gw = jnp.maximum(gw, 1.0)
    gh = jnp.maximum(gh, 1.0)
    t0 = ((gcx - acx) / aw) / 0.1
    t1 = ((gcy - acy) / ah) / 0.1
    t2 = jnp.log(gw / aw) / 0.2
    t3 = jnp.log(gh / ah) / 0.2
    reg = reg_ref[0, 0]
    d0 = jnp.abs(t0 - reg[0:1, :])
    d1 = jnp.abs(t1 - reg[1:2, :])
    d2 = jnp.abs(t2 - reg[2:3, :])
    d3 = jnp.abs(t3 - reg[3:4, :])

    def smooth_l1(d):
        return jnp.where(d < 1.0 / 9.0, 0.5 * 9.0 * (d * d), d - 0.5 / 9.0)

    rl = smooth_l1(d0) + smooth_l1(d1) + smooth_l1(d2) + smooth_l1(d3)
    reg_part = jnp.sum(jnp.where(pos, rl, 0.0))

    glab_col = jnp.transpose(glab, (1, 0))

    cls = jnp.clip(cls_ref[0], 0.0001, 1.0 - 0.0001)
    logm = jnp.log(1.0 - cls)
    nege = (0.75 * (cls * cls)) * logm
    tot_c = jnp.dot(keep.astype(jnp.bfloat16), nege.astype(jnp.bfloat16),
                    preferred_element_type=jnp.float32)
    ci = lax.broadcasted_iota(jnp.int32, (bn, c), 1)
    g_col = jnp.sum(jnp.where(ci == glab_col.astype(jnp.int32), cls, 0.0),
                    axis=1, keepdims=True)
    g = jnp.transpose(g_col, (1, 0))
    pos_e = (0.25 * ((1.0 - g) * (1.0 - g))) * (-jnp.log(g))
    neg_e = (0.75 * (g * g)) * (-jnp.log(1.0 - g))
    corr = jnp.where(pos, pos_e - neg_e, 0.0)
    cls_part = jnp.sum(corr) - jnp.sum(tot_c)

    cls_o[...] += jnp.full(cls_o.shape, cls_part, jnp.float32)
    npos_o[...] += jnp.full(npos_o.shape, npos_part, jnp.float32)
    reg_o[...] += jnp.full(reg_o.shape, reg_part, jnp.float32)


@jax.jit
def kernel(classifications, regressions, anchors, annotations):
    b, n, c = classifications.shape
    m = annotations.shape[1]
    bn = _BN
    nb = n // bn
    per_w = (b * n) // _NW

    anc_pad = jnp.pad(anchors.reshape(_NW, per_w, 4),
                      ((0, 0), (0, _CH - per_w), (0, 0))).transpose(0, 2, 1)
    ann_sc = annotations.transpose(0, 2, 1).reshape(b, 5 * m)
    scm = _sc_match(anc_pad, ann_sc, m)
    scm = scm[:, :, :per_w].reshape(b, _NW // b, 6, per_w)
    scm = scm.transpose(0, 2, 1, 3).reshape(b, 6, nb, bn)
    scm = scm.transpose(0, 2, 1, 3)

    anc_t = anchors.reshape(b, nb, bn, 4).transpose(0, 1, 3, 2)
    reg_t = regressions.reshape(b, nb, bn, 4).transpose(0, 1, 3, 2)

    body = functools.partial(_tc_body, bn=bn, c=c)
    out_sds = jax.ShapeDtypeStruct((b, 1, 128), jnp.float32)
    cls_s, npos, reg_s = pl.pallas_call(
        body,
        grid=(b, nb),
        in_specs=[
            pl.BlockSpec((1, bn, c), lambda bb, ii: (bb, ii, 0)),
            pl.BlockSpec((1, 1, 4, bn), lambda bb, ii: (bb, ii, 0, 0)),
            pl.BlockSpec((1, 1, 4, bn), lambda bb, ii: (bb, ii, 0, 0)),
            pl.BlockSpec((1, 1, 6, bn), lambda bb, ii: (bb, ii, 0, 0)),
        ],
        out_specs=[
            pl.BlockSpec((1, 1, 128), lambda bb, ii: (bb, 0, 0)),
            pl.BlockSpec((1, 1, 128), lambda bb, ii: (bb, 0, 0)),
            pl.BlockSpec((1, 1, 128), lambda bb, ii: (bb, 0, 0)),
        ],
        out_shape=[out_sds, out_sds, out_sds],
    )(classifications, reg_t, anc_t, scm)

    cls_s = cls_s[:, 0, 0]
    npos = npos[:, 0, 0]
    reg_s = reg_s[:, 0, 0]
    cls_tot = jnp.where(npos > 0, cls_s / jnp.maximum(npos, 1.0), 0.0)
    reg_tot = jnp.where(npos > 0, reg_s / jnp.maximum(4.0 * npos, 1.0), 0.0)
    return jnp.mean(cls_tot), jnp.mean(reg_tot)


# speedup vs baseline: 1.4912x; 1.0019x over previous
import functools

import jax
import jax.numpy as jnp
from jax import lax
from jax.experimental import pallas as pl
from jax.experimental.pallas import tpu as pltpu
from jax.experimental.pallas import tpu_sc as plsc

_BN = 4000
_CH = 2512
_NW = 32


def _sc_body(anc_hbm, ann_hbm, out_hbm,
             ax1v, ay1v, ax2v, ay2v, annv,
             omaxv, og1v, og2v, og3v, og4v, olabv, *, m):
    wid = lax.axis_index("s") * 2 + lax.axis_index("c")
    b = wid // 8
    pltpu.sync_copy(anc_hbm.at[wid, 0], ax1v)
    pltpu.sync_copy(anc_hbm.at[wid, 1], ay1v)
    pltpu.sync_copy(anc_hbm.at[wid, 2], ax2v)
    pltpu.sync_copy(anc_hbm.at[wid, 3], ay2v)
    pltpu.sync_copy(ann_hbm.at[b], annv)

    def field(k):
        vs = [annv[pl.ds(k * m + h * 16, 16)] for h in range(m // 16)]
        return [vs[mm // 16][mm % 16] for mm in range(m)]

    bx1s = field(0)
    by1s = field(1)
    bx2s = field(2)
    by2s = field(3)
    labs = field(4)
    areab = [(bx2s[mm] - bx1s[mm]) * (by2s[mm] - by1s[mm]) for mm in range(m)]

    @plsc.parallel_loop(0, _CH // 16, unroll=4)
    def group(j):
        sl = pl.ds(j * 16, 16)
        ax1 = ax1v[sl]
        ay1 = ay1v[sl]
        ax2 = ax2v[sl]
        ay2 = ay2v[sl]
        area_a = (ax2 - ax1) * (ay2 - ay1)
        best = jnp.full((16,), -2.0, jnp.float32)
        barg = jnp.zeros((16,), jnp.int32)
        for mm in range(m):
            iw = jnp.maximum(jnp.minimum(ax2, bx2s[mm]) - jnp.maximum(ax1, bx1s[mm]), 0.0)
            ih = jnp.maximum(jnp.minimum(ay2, by2s[mm]) - jnp.maximum(ay1, by1s[mm]), 0.0)
            inter = iw * ih
            union = jnp.maximum(area_a + areab[mm] - inter, 1e-8)
            iou = inter / union
            iou = jnp.where(labs[mm] != -1.0, iou, -1.0)
            upd = iou > best
            barg = jnp.where(upd, mm, barg)
            best = jnp.where(upd, iou, best)
        omaxv[sl] = best
        og1v[sl] = plsc.load_gather(annv, [barg])
        og2v[sl] = plsc.load_gather(annv, [barg + m])
        og3v[sl] = plsc.load_gather(annv, [barg + 2 * m])
        og4v[sl] = plsc.load_gather(annv, [barg + 3 * m])
        olabv[sl] = plsc.load_gather(annv, [barg + 4 * m])

    pltpu.sync_copy(omaxv, out_hbm.at[wid, 0])
    pltpu.sync_copy(og1v, out_hbm.at[wid, 1])
    pltpu.sync_copy(og2v, out_hbm.at[wid, 2])
    pltpu.sync_copy(og3v, out_hbm.at[wid, 3])
    pltpu.sync_copy(og4v, out_hbm.at[wid, 4])
    pltpu.sync_copy(olabv, out_hbm.at[wid, 5])


def _sc_match(anc_pad, annotations, m):
    mesh = plsc.VectorSubcoreMesh(core_axis_name="c", subcore_axis_name="s")
    f32 = jnp.float32
    kern = functools.partial(
        pl.kernel,
        mesh=mesh,
        compiler_params=pltpu.CompilerParams(needs_layout_passes=False),
        out_type=jax.ShapeDtypeStruct((_NW, 6, _CH), f32),
        scratch_types=[
            pltpu.VMEM((_CH,), f32), pltpu.VMEM((_CH,), f32),
            pltpu.VMEM((_CH,), f32), pltpu.VMEM((_CH,), f32),
            pltpu.VMEM((5 * m,), f32),
            pltpu.VMEM((_CH,), f32), pltpu.VMEM((_CH,), f32),
            pltpu.VMEM((_CH,), f32), pltpu.VMEM((_CH,), f32),
            pltpu.VMEM((_CH,), f32), pltpu.VMEM((_CH,), f32),
        ],
    )(functools.partial(_sc_body, m=m))
    return kern(anc_pad, annotations)


def _tc_body(cls_ref, reg_ref, anc_ref, sc_ref, cls_o, npos_o, reg_o, *, bn, c):
    i = pl.program_id(1)

    @pl.when(i == 0)
    def _init():
        cls_o[...] = jnp.zeros_like(cls_o)
        npos_o[...] = jnp.zeros_like(npos_o)
        reg_o[...] = jnp.zeros_like(reg_o)

    sc = sc_ref[0, 0]
    iou_max = sc[0:1, :]
    gx1 = sc[1:2, :]
    gy1 = sc[2:3, :]
    gx2 = sc[3:4, :]
    gy2 = sc[4:5, :]
    glab = sc[5:6, :]
    anc = anc_ref[0, 0]
    ax1 = anc[0:1, :]
    ay1 = anc[1:2, :]
    ax2 = anc[2:3, :]
    ay2 = anc[3:4, :]

    pos = iou_max >= 0.5
    keep = jnp.logical_or(iou_max < 0.4, pos)
    npos_part = jnp.sum(pos.astype(jnp.float32))

    aw = ax2 - ax1
    ah = ay2 - ay1
    acx = ax1 + 0.5 * aw
    acy = ay1 + 0.5 * ah
    gw = gx2 - gx1
    gh = gy2 - gy1
    gcx = gx1 + 0.5 * gw
    gcy = gy1 + 0.5 * gh
    gw = jnp.maximum(gw, 1.0)
    gh = jnp.maximum(gh, 1.0)
    t0 = ((gcx - acx) / aw) / 0.1
    t1 = ((gcy - acy) / ah) / 0.1
    t2 = jnp.log(gw / aw) / 0.2
    t3 = jnp.log(gh / ah) / 0.2
    reg = reg_ref[0, 0]
    d0 = jnp.abs(t0 - reg[0:1, :])
    d1 = jnp.abs(t1 - reg[1:2, :])
    d2 = jnp.abs(t2 - reg[2:3, :])
    d3 = jnp.abs(t3 - reg[3:4, :])

    def smooth_l1(d):
        return jnp.where(d < 1.0 / 9.0, 0.5 * 9.0 * (d * d), d - 0.5 / 9.0)

    rl = smooth_l1(d0) + smooth_l1(d1) + smooth_l1(d2) + smooth_l1(d3)
    reg_part = jnp.sum(jnp.where(pos, rl, 0.0))

    glab_col = jnp.transpose(glab, (1, 0))

    cls = jnp.clip(cls_ref[0], 0.0001, 1.0 - 0.0001)
    logm = jnp.log(1.0 - cls)
    nege = (0.75 * (cls * cls)) * logm
    tot_c = jnp.dot(keep.astype(jnp.bfloat16), nege.astype(jnp.bfloat16),
                    preferred_element_type=jnp.float32)
    ci = lax.broadcasted_iota(jnp.int32, (bn, c), 1)
    g_col = jnp.sum(jnp.where(ci == glab_col.astype(jnp.int32), cls, 0.0),
                    axis=1, keepdims=True)
    g = jnp.transpose(g_col, (1, 0))
    pos_e = (0.25 * ((1.0 - g) * (1.0 - g))) * (-jnp.log(g))
    neg_e = (0.75 * (g * g)) * (-jnp.log(1.0 - g))
    corr = jnp.where(pos, pos_e - neg_e, 0.0)
    cls_part = jnp.sum(corr) - jnp.sum(tot_c)

    cls_o[...] += jnp.full(cls_o.shape, cls_part, jnp.float32)
    npos_o[...] += jnp.full(npos_o.shape, npos_part, jnp.float32)
    reg_o[...] += jnp.full(reg_o.shape, reg_part, jnp.float32)


@jax.jit
def kernel(classifications, regressions, anchors, annotations):
    b, n, c = classifications.shape
    m = annotations.shape[1]
    bn = _BN
    nb = n // bn
    per_w = (b * n) // _NW

    anc_pad = jnp.pad(anchors.reshape(_NW, per_w, 4),
                      ((0, 0), (0, _CH - per_w), (0, 0))).transpose(0, 2, 1)
    ann_sc = annotations.transpose(0, 2, 1).reshape(b, 5 * m)
    scm = _sc_match(anc_pad, ann_sc, m)
    scm = scm[:, :, :per_w].reshape(b, _NW // b, 6, per_w)
    scm = scm.transpose(0, 2, 1, 3).reshape(b, 6, nb, bn)
    scm = scm.transpose(0, 2, 1, 3)

    anc_t = anchors.reshape(b, nb, bn, 4).transpose(0, 1, 3, 2)
    reg_t = regressions.reshape(b, nb, bn, 4).transpose(0, 1, 3, 2)

    body = functools.partial(_tc_body, bn=bn, c=c)
    out_sds = jax.ShapeDtypeStruct((b, 1, 128), jnp.float32)
    cls_s, npos, reg_s = pl.pallas_call(
        body,
        grid=(b, nb),
        in_specs=[
            pl.BlockSpec((1, bn, c), lambda bb, ii: (bb, ii, 0)),
            pl.BlockSpec((1, 1, 4, bn), lambda bb, ii: (bb, ii, 0, 0)),
            pl.BlockSpec((1, 1, 4, bn), lambda bb, ii: (bb, ii, 0, 0)),
            pl.BlockSpec((1, 1, 6, bn), lambda bb, ii: (bb, ii, 0, 0)),
        ],
        out_specs=[
            pl.BlockSpec((1, 1, 128), lambda bb, ii: (bb, 0, 0)),
            pl.BlockSpec((1, 1, 128), lambda bb, ii: (bb, 0, 0)),
            pl.BlockSpec((1, 1, 128), lambda bb, ii: (bb, 0, 0)),
        ],
        out_shape=[out_sds, out_sds, out_sds],
    )(classifications, reg_t, anc_t, scm)

    cls_s = cls_s[:, 0, 0]
    npos = npos[:, 0, 0]
    reg_s = reg_s[:, 0, 0]
    cls_tot = jnp.where(npos > 0, cls_s / jnp.maximum(npos, 1.0), 0.0)
    reg_tot = jnp.where(npos > 0, reg_s / jnp.maximum(4.0 * npos, 1.0), 0.0)
    return jnp.mean(cls_tot), jnp.mean(reg_tot)
